# baseline (speedup 1.0000x reference)
import functools

import jax
import jax.numpy as jnp
from jax import lax
from jax.experimental import pallas as pl
from jax.experimental.pallas import tpu as pltpu
from jax.experimental.pallas import tpu_sc as plsc

EMBED_DIM = 32
PRED_DIM = 128
BATCH = 16384

_info = plsc.get_sparse_core_info()
_NC, _NS = _info.num_cores, _info.num_subcores
_NW = _NC * _NS
_BPW = BATCH // _NW

_mesh = plsc.VectorSubcoreMesh(core_axis_name="c", subcore_axis_name="s")


@functools.partial(
    pl.kernel,
    mesh=_mesh,
    out_type=jax.ShapeDtypeStruct((BATCH, EMBED_DIM), jnp.float32),
    compiler_params=pltpu.CompilerParams(use_tc_tiling_on_sc=False),
    scratch_types=[
        pltpu.VMEM((_BPW,), jnp.int32),
        pltpu.VMEM((_BPW, EMBED_DIM), jnp.float32),
        pltpu.SemaphoreType.DMA,
    ],
)
def _sc_gather(table_hbm, idx_hbm, out_hbm, idx_v, rows_v, sem):
    wid = lax.axis_index("s") * _NC + lax.axis_index("c")
    base = wid * _BPW
    pltpu.sync_copy(idx_hbm.at[pl.ds(base, _BPW)], idx_v)
    pltpu.async_copy(table_hbm.at[idx_v], rows_v, sem).wait()
    pltpu.sync_copy(rows_v, out_hbm.at[pl.ds(base, _BPW)])


_ROWS_BLK = 2048


def _tc_head_body(pred_ref, emb_ref, w_ref, b_ref, out_ref):
    wp = w_ref[:PRED_DIM, :]
    we = w_ref[PRED_DIM:, :]
    acc = jnp.dot(pred_ref[...], wp, preferred_element_type=jnp.float32)
    acc = acc + jnp.dot(emb_ref[...], we, preferred_element_type=jnp.float32)
    out_ref[...] = acc + b_ref[...]


def _tc_head(predictors, emb, W, b):
    grid = (BATCH // _ROWS_BLK,)
    return pl.pallas_call(
        _tc_head_body,
        grid=grid,
        in_specs=[
            pl.BlockSpec((_ROWS_BLK, PRED_DIM), lambda i: (i, 0)),
            pl.BlockSpec((_ROWS_BLK, EMBED_DIM), lambda i: (i, 0)),
            pl.BlockSpec((PRED_DIM + EMBED_DIM, 1), lambda i: (0, 0)),
            pl.BlockSpec((1,), lambda i: (0,)),
        ],
        out_specs=pl.BlockSpec((_ROWS_BLK, 1), lambda i: (i, 0)),
        out_shape=jax.ShapeDtypeStruct((BATCH, 1), jnp.float32),
    )(predictors, emb, W, b)


def kernel(predictors, encoding, emb_table, W, b):
    emb = _sc_gather(emb_table, encoding)
    return _tc_head(predictors, emb, W, b)


# speedup vs baseline: 5.1834x; 5.1834x over previous
import functools

import jax
import jax.numpy as jnp
from jax import lax
from jax.experimental import pallas as pl
from jax.experimental.pallas import tpu as pltpu
from jax.experimental.pallas import tpu_sc as plsc

EMBED_DIM = 32
PRED_DIM = 128
BATCH = 16384
NUM_EMB = 1000000

_info = plsc.get_sparse_core_info()
_NC, _NS = _info.num_cores, _info.num_subcores
_NW = _NC * _NS
_BPW = BATCH // _NW

_mesh = plsc.VectorSubcoreMesh(core_axis_name="c", subcore_axis_name="s")


@functools.partial(
    pl.kernel,
    mesh=_mesh,
    out_type=jax.ShapeDtypeStruct((BATCH,), jnp.float32),
    compiler_params=pltpu.CompilerParams(use_tc_tiling_on_sc=False),
    scratch_types=[
        pltpu.VMEM((_BPW,), jnp.int32),
        pltpu.VMEM((_BPW,), jnp.float32),
        pltpu.SemaphoreType.DMA,
    ],
)
def _sc_gather(table_dot_hbm, idx_hbm, out_hbm, idx_v, vals_v, sem):
    wid = lax.axis_index("s") * _NC + lax.axis_index("c")
    base = wid * _BPW
    pltpu.sync_copy(idx_hbm.at[pl.ds(base, _BPW)], idx_v)
    pltpu.async_copy(table_dot_hbm.at[idx_v], vals_v, sem).wait()
    pltpu.sync_copy(vals_v, out_hbm.at[pl.ds(base, _BPW)])


_DOT_BLK = 16384


def _table_dot_body(tbl_ref, we_ref, out_ref):
    out_ref[...] = jnp.sum(tbl_ref[...] * we_ref[...], axis=0)


def _table_dot(table_t, W):
    we = W[PRED_DIM:, :]
    grid = (pl.cdiv(NUM_EMB, _DOT_BLK),)
    return pl.pallas_call(
        _table_dot_body,
        grid=grid,
        in_specs=[
            pl.BlockSpec((EMBED_DIM, _DOT_BLK), lambda i: (0, i)),
            pl.BlockSpec((EMBED_DIM, 1), lambda i: (0, 0)),
        ],
        out_specs=pl.BlockSpec((_DOT_BLK,), lambda i: (i,)),
        out_shape=jax.ShapeDtypeStruct((NUM_EMB,), jnp.float32),
    )(table_t, we)


_ROWS_BLK = 2048


def _head_body(pred_ref, g_ref, w_ref, b_ref, out_ref):
    acc = jnp.dot(pred_ref[...], w_ref[...], preferred_element_type=jnp.float32)
    out_ref[...] = acc + g_ref[...][:, None] + b_ref[...]


def _head(predictors, gathered, W, b):
    wp = W[:PRED_DIM, :]
    grid = (BATCH // _ROWS_BLK,)
    return pl.pallas_call(
        _head_body,
        grid=grid,
        in_specs=[
            pl.BlockSpec((_ROWS_BLK, PRED_DIM), lambda i: (i, 0)),
            pl.BlockSpec((_ROWS_BLK,), lambda i: (i,)),
            pl.BlockSpec((PRED_DIM, 1), lambda i: (0, 0)),
            pl.BlockSpec((1,), lambda i: (0,)),
        ],
        out_specs=pl.BlockSpec((_ROWS_BLK, 1), lambda i: (i, 0)),
        out_shape=jax.ShapeDtypeStruct((BATCH, 1), jnp.float32),
    )(predictors, gathered, wp, b)


def kernel(predictors, encoding, emb_table, W, b):
    table_dot = _table_dot(emb_table.T, W)
    gathered = _sc_gather(table_dot, encoding)
    return _head(predictors, gathered, W, b)


# speedup vs baseline: 6.1912x; 1.1944x over previous
import functools

import jax
import jax.numpy as jnp
from jax import lax
from jax.experimental import pallas as pl
from jax.experimental.pallas import tpu as pltpu
from jax.experimental.pallas import tpu_sc as plsc

EMBED_DIM = 32
PRED_DIM = 128
BATCH = 16384
NUM_EMB = 1000000

_info = plsc.get_sparse_core_info()
_NC, _NS = _info.num_cores, _info.num_subcores
_NW = _NC * _NS
_BPW = BATCH // _NW

_mesh = plsc.VectorSubcoreMesh(core_axis_name="c", subcore_axis_name="s")


@functools.partial(
    pl.kernel,
    mesh=_mesh,
    out_type=jax.ShapeDtypeStruct((BATCH,), jnp.float32),
    compiler_params=pltpu.CompilerParams(use_tc_tiling_on_sc=False),
    scratch_types=[
        pltpu.VMEM((_BPW,), jnp.int32),
        pltpu.VMEM((_BPW,), jnp.float32),
        pltpu.SemaphoreType.DMA,
    ],
)
def _sc_gather(table_dot_hbm, idx_hbm, out_hbm, idx_v, vals_v, sem):
    wid = lax.axis_index("s") * _NC + lax.axis_index("c")
    base = wid * _BPW
    pltpu.sync_copy(idx_hbm.at[pl.ds(base, _BPW)], idx_v)
    pltpu.async_copy(table_dot_hbm.at[idx_v], vals_v, sem).wait()
    pltpu.sync_copy(vals_v, out_hbm.at[pl.ds(base, _BPW)])


_DOT_BLK = 32768


def _table_dot_body(tbl_ref, we_ref, out_ref):
    out_ref[...] = jnp.sum(tbl_ref[...] * we_ref[...], axis=0)


def _table_dot(table_t, W):
    we = W[PRED_DIM:, :]
    grid = (pl.cdiv(NUM_EMB, _DOT_BLK),)
    return pl.pallas_call(
        _table_dot_body,
        grid=grid,
        in_specs=[
            pl.BlockSpec((EMBED_DIM, _DOT_BLK), lambda i: (0, i)),
            pl.BlockSpec((EMBED_DIM, 1), lambda i: (0, 0)),
        ],
        out_specs=pl.BlockSpec((_DOT_BLK,), lambda i: (i,)),
        out_shape=jax.ShapeDtypeStruct((NUM_EMB,), jnp.float32),
    )(table_t, we)


_ROWS_BLK = 2048


def _head_body(pred_ref, g_ref, w_ref, b_ref, out_ref):
    acc = jnp.dot(pred_ref[...], w_ref[...], preferred_element_type=jnp.float32)
    out_ref[...] = acc + g_ref[...][:, None] + b_ref[...]


def _head(predictors, gathered, W, b):
    wp = W[:PRED_DIM, :]
    grid = (BATCH // _ROWS_BLK,)
    return pl.pallas_call(
        _head_body,
        grid=grid,
        in_specs=[
            pl.BlockSpec((_ROWS_BLK, PRED_DIM), lambda i: (i, 0)),
            pl.BlockSpec((_ROWS_BLK,), lambda i: (i,)),
            pl.BlockSpec((PRED_DIM, 1), lambda i: (0, 0)),
            pl.BlockSpec((1,), lambda i: (0,)),
        ],
        out_specs=pl.BlockSpec((_ROWS_BLK, 1), lambda i: (i, 0)),
        out_shape=jax.ShapeDtypeStruct((BATCH, 1), jnp.float32),
    )(predictors, gathered, wp, b)


def kernel(predictors, encoding, emb_table, W, b):
    table_dot = _table_dot(emb_table.T, W)
    gathered = _sc_gather(table_dot, encoding)
    return _head(predictors, gathered, W, b)


# speedup vs baseline: 6.7060x; 1.0832x over previous
import functools

import jax
import jax.numpy as jnp
from jax import lax
from jax.experimental import pallas as pl
from jax.experimental.pallas import tpu as pltpu
from jax.experimental.pallas import tpu_sc as plsc

EMBED_DIM = 32
PRED_DIM = 128
BATCH = 16384
NUM_EMB = 1000000

_info = plsc.get_sparse_core_info()
_NC, _NS = _info.num_cores, _info.num_subcores
_NW = _NC * _NS
_BPW = BATCH // _NW

_mesh = plsc.VectorSubcoreMesh(core_axis_name="c", subcore_axis_name="s")


@functools.partial(
    pl.kernel,
    mesh=_mesh,
    out_type=jax.ShapeDtypeStruct((BATCH,), jnp.float32),
    compiler_params=pltpu.CompilerParams(use_tc_tiling_on_sc=False),
    scratch_types=[
        pltpu.VMEM((_BPW,), jnp.int32),
        pltpu.VMEM((_BPW,), jnp.float32),
        pltpu.SemaphoreType.DMA,
    ],
)
def _sc_gather(table_dot_hbm, idx_hbm, out_hbm, idx_v, vals_v, sem):
    wid = lax.axis_index("s") * _NC + lax.axis_index("c")
    base = wid * _BPW
    pltpu.sync_copy(idx_hbm.at[pl.ds(base, _BPW)], idx_v)
    pltpu.async_copy(table_dot_hbm.at[idx_v], vals_v, sem).wait()
    pltpu.sync_copy(vals_v, out_hbm.at[pl.ds(base, _BPW)])


_DOT_BLK = 65536


def _table_dot_body(tbl_ref, we_ref, out_ref):
    out_ref[...] = jnp.sum(tbl_ref[...] * we_ref[...], axis=0)


def _table_dot(table_t, W):
    we = W[PRED_DIM:, :]
    grid = (pl.cdiv(NUM_EMB, _DOT_BLK),)
    return pl.pallas_call(
        _table_dot_body,
        grid=grid,
        in_specs=[
            pl.BlockSpec((EMBED_DIM, _DOT_BLK), lambda i: (0, i)),
            pl.BlockSpec((EMBED_DIM, 1), lambda i: (0, 0)),
        ],
        out_specs=pl.BlockSpec((_DOT_BLK,), lambda i: (i,)),
        out_shape=jax.ShapeDtypeStruct((NUM_EMB,), jnp.float32),
    )(table_t, we)


_ROWS_BLK = 2048


def _head_body(pred_ref, g_ref, w_ref, b_ref, out_ref):
    acc = jnp.dot(pred_ref[...], w_ref[...], preferred_element_type=jnp.float32)
    out_ref[...] = acc + g_ref[...][:, None] + b_ref[...]


def _head(predictors, gathered, W, b):
    wp = W[:PRED_DIM, :]
    grid = (BATCH // _ROWS_BLK,)
    return pl.pallas_call(
        _head_body,
        grid=grid,
        in_specs=[
            pl.BlockSpec((_ROWS_BLK, PRED_DIM), lambda i: (i, 0)),
            pl.BlockSpec((_ROWS_BLK,), lambda i: (i,)),
            pl.BlockSpec((PRED_DIM, 1), lambda i: (0, 0)),
            pl.BlockSpec((1,), lambda i: (0,)),
        ],
        out_specs=pl.BlockSpec((_ROWS_BLK, 1), lambda i: (i, 0)),
        out_shape=jax.ShapeDtypeStruct((BATCH, 1), jnp.float32),
    )(predictors, gathered, wp, b)


def kernel(predictors, encoding, emb_table, W, b):
    table_dot = _table_dot(emb_table.T, W)
    gathered = _sc_gather(table_dot, encoding)
    return _head(predictors, gathered, W, b)
